# Initial kernel scaffold; baseline (speedup 1.0000x reference)
import jax
import jax.numpy as jnp
from jax.experimental import pallas as pl


def kernel(x, edge_index, edge_weight, W1, b1, g1, be1, W2, b2, g2, be2, w_ih1, w_hh1, b_ih1, b_hh1, w_ih2, w_hh2, b_ih2, b_hh2, W_lin, b_lin):
    raise NotImplementedError("write your pallas kernel here")


# speedup vs baseline: 3.7602x; 3.7602x over previous
import functools

import jax
import jax.numpy as jnp
from jax import lax
from jax.experimental import pallas as pl
from jax.experimental.pallas import tpu as pltpu
from jax.experimental.pallas import tpu_sc as plsc

N = 10062
E = 321984
IN = 128
HID = 256
OUT = 128
EPS = 1e-5

NP = 10240
NBLK = 128
NG = NP // NBLK
EW_ = 112
EROWS = 3072
EPAD = EROWS * EW_
TROWS = EROWS // 16
AROWS = EROWS // 32
NPT = NP // 16
CH = 16

_f32 = jnp.float32
_i32 = jnp.int32

_WCHUNKS = (112, 112, 112, 112, 112, 80)


def _zero16():
    return jnp.zeros((16,), _f32)


@functools.cache
def _sc_degree_call():
    mesh = plsc.VectorSubcoreMesh(core_axis_name="c", subcore_axis_name="s")
    return functools.partial(
        pl.kernel,
        mesh=mesh,
        out_type=jax.ShapeDtypeStruct((2, NP), _f32),
        scratch_types=[
            pltpu.VMEM((AROWS, EW_), _i32),
            pltpu.VMEM((AROWS, EW_), _f32),
            pltpu.VMEM((NPT,), _f32),
            pltpu.VMEM_SHARED((NP,), _f32),
        ],
    )(_sc_degree_body)


def _sc_degree_body(dst_hbm, ew_hbm, out_hbm, dst_v, ew_v, zv, acc_sh):
    c = lax.axis_index("c")
    s = lax.axis_index("s")

    def zb(i, carry):
        zv[pl.ds(i * 16, 16)] = _zero16()
        return carry

    lax.fori_loop(0, NPT // 16, zb, 0)
    pltpu.sync_copy(zv, acc_sh.at[pl.ds(s * NPT, NPT)])
    plsc.subcore_barrier()

    base = (c * 16 + s) * AROWS
    pltpu.sync_copy(dst_hbm.at[pl.ds(base, AROWS)], dst_v)
    pltpu.sync_copy(ew_hbm.at[pl.ds(base, AROWS)], ew_v)

    def body(g, carry):
        pltpu.sync_copy(ew_v.at[g], acc_sh.at[dst_v.at[g]], add=True)
        return carry

    lax.fori_loop(0, AROWS, body, 0)
    plsc.subcore_barrier()
    pltpu.sync_copy(acc_sh.at[pl.ds(s * NPT, NPT)], out_hbm.at[c, pl.ds(s * NPT, NPT)])


@functools.cache
def _sc_spmm_call():
    mesh = plsc.VectorSubcoreMesh(core_axis_name="c", subcore_axis_name="s")
    return functools.partial(
        pl.kernel,
        mesh=mesh,
        out_type=jax.ShapeDtypeStruct((2 * NP, 128), _f32),
        scratch_types=[
            pltpu.VMEM((CH, EW_), _i32),
            pltpu.VMEM((CH, EW_), _i32),
            pltpu.VMEM((CH, EW_), _f32),
            pltpu.VMEM((EW_, 128), _f32),
            pltpu.VMEM_SHARED((NP, 128), _f32),
            pltpu.SemaphoreType.DMA,
        ],
    )(_sc_spmm_body)


def _sc_spmm_body(src_hbm, dst_hbm, ew_hbm, xs_hbm, out_hbm,
                  src_v, dst_v, ew_v, rbuf, acc_sh, sem):
    c = lax.axis_index("c")
    s = lax.axis_index("s")
    coff = c * NP

    def zb(i, carry):
        for f in range(8):
            rbuf[i, pl.ds(f * 16, 16)] = _zero16()
        return carry

    lax.fori_loop(0, EW_, zb, 0)
    off = 0
    for sz in _WCHUNKS:
        pltpu.sync_copy(rbuf.at[pl.ds(0, sz)],
                        acc_sh.at[pl.ds(s * NPT + off, sz)])
        off += sz
    plsc.subcore_barrier()

    base = s * TROWS

    def outer(t, carry):
        row0 = base + t * CH
        pltpu.sync_copy(src_hbm.at[pl.ds(row0, CH)], src_v)
        pltpu.sync_copy(dst_hbm.at[pl.ds(row0, CH)], dst_v)
        pltpu.sync_copy(ew_hbm.at[pl.ds(row0, CH)], ew_v)

        def adj(g, carry2):
            for j in range(EW_ // 16):
                sl = src_v[g, pl.ds(j * 16, 16)]
                src_v[g, pl.ds(j * 16, 16)] = sl + coff
            return carry2

        lax.fori_loop(0, CH, adj, 0)

        def body(g, carry2):
            pltpu.async_copy(xs_hbm.at[src_v.at[g]], rbuf, sem).wait()
            for j in range(EW_ // 16):
                wrow = ew_v[g, pl.ds(j * 16, 16)]
                for l in range(16):
                    i = j * 16 + l
                    wv = wrow[l]
                    for f in range(8):
                        rbuf[i, pl.ds(f * 16, 16)] = rbuf[i, pl.ds(f * 16, 16)] * wv
            pltpu.sync_copy(rbuf, acc_sh.at[dst_v.at[g]], add=True)
            return carry2

        lax.fori_loop(0, CH, body, 0)
        return carry

    lax.fori_loop(0, TROWS // CH, outer, 0)
    plsc.subcore_barrier()

    off = 0
    for sz in _WCHUNKS:
        pltpu.sync_copy(acc_sh.at[pl.ds(s * NPT + off, sz)],
                        out_hbm.at[pl.ds(coff + s * NPT + off, sz)])
        off += sz


def sc_degree(dst2d, ew2d):
    return _sc_degree_call()(dst2d, ew2d)


def sc_spmm(src2d, dst2d, ew2d, xs):
    return _sc_spmm_call()(src2d, dst2d, ew2d, xs)


def _dis_block(degref, i):
    d = degref[0, pl.ds(i * NBLK, NBLK)] + degref[1, pl.ds(i * NBLK, NBLK)] + 1.0
    return lax.rsqrt(d)


def tc_xw_scale(x_p, W, deg2):
    kdim = x_p.shape[1]

    def body(xref, wref, degref, oref):
        i = pl.program_id(0)
        dis = _dis_block(degref, i)
        xw = jnp.dot(xref[...], wref[...], preferred_element_type=_f32)
        oref[...] = xw * dis[:, None]

    return pl.pallas_call(
        body,
        grid=(NG, 2),
        in_specs=[
            pl.BlockSpec((NBLK, kdim), lambda i, c: (i, 0)),
            pl.BlockSpec((kdim, 128), lambda i, c: (0, c)),
            pl.BlockSpec((2, NP), lambda i, c: (0, 0)),
        ],
        out_specs=pl.BlockSpec((NBLK, 128), lambda i, c: (c * NG + i, 0)),
        out_shape=jax.ShapeDtypeStruct((2 * NP, 128), _f32),
    )(x_p, W, deg2)


def tc_conv_post(s_parts, xs_parts, deg2, b):
    def body(saref, sbref, xaref, xbref, degref, bref, href, statref, acc):
        i = pl.program_id(0)
        dis = _dis_block(degref, i)[:, None]
        ha = dis * (saref[...] + xaref[...]) + bref[:, :128]
        hb = dis * (sbref[...] + xbref[...]) + bref[:, 128:]
        h = jax.nn.relu(jnp.concatenate([ha, hb], axis=1))
        href[...] = h
        rid = i * NBLK + lax.broadcasted_iota(_i32, (NBLK, 1), 0)
        hm = jnp.where(rid < N, h, 0.0)
        blk = jnp.concatenate([jnp.sum(hm, 0, keepdims=True),
                               jnp.sum(hm * hm, 0, keepdims=True)], axis=0)

        @pl.when(i == 0)
        def _():
            acc[...] = jnp.zeros((2, HID), _f32)

        acc[...] += blk

        @pl.when(i == NG - 1)
        def _():
            statref[...] = acc[...]

    return pl.pallas_call(
        body,
        grid=(NG,),
        in_specs=[
            pl.BlockSpec((NBLK, 128), lambda i: (i, 0)),
            pl.BlockSpec((NBLK, 128), lambda i: (NG + i, 0)),
            pl.BlockSpec((NBLK, 128), lambda i: (i, 0)),
            pl.BlockSpec((NBLK, 128), lambda i: (NG + i, 0)),
            pl.BlockSpec((2, NP), lambda i: (0, 0)),
            pl.BlockSpec((1, HID), lambda i: (0, 0)),
        ],
        out_specs=[
            pl.BlockSpec((NBLK, HID), lambda i: (i, 0)),
            pl.BlockSpec((2, HID), lambda i: (0, 0)),
        ],
        out_shape=[
            jax.ShapeDtypeStruct((NP, HID), _f32),
            jax.ShapeDtypeStruct((2, HID), _f32),
        ],
        scratch_shapes=[pltpu.VMEM((2, HID), _f32)],
    )(s_parts, s_parts, xs_parts, xs_parts, deg2, b)


def _bn_apply(h, stats, gref, beref):
    mean = stats[0:1, :] * (1.0 / N)
    var = stats[1:2, :] * (1.0 / N) - mean * mean
    scale = gref[...] * lax.rsqrt(var + EPS)
    shift = beref[...] - mean * scale
    return h * scale + shift


def tc_bn_matmul(h, stats, g, be, W2, deg2):
    def body(href, statref, gref, beref, wref, degref, rref, oref):
        i = pl.program_id(0)
        r = _bn_apply(href[...], statref[...], gref, beref)
        rref[...] = r
        dis = _dis_block(degref, i)
        oref[...] = jnp.dot(r, wref[...], preferred_element_type=_f32) * dis[:, None]

    return pl.pallas_call(
        body,
        grid=(NG, 2),
        in_specs=[
            pl.BlockSpec((NBLK, HID), lambda i, c: (i, 0)),
            pl.BlockSpec((2, HID), lambda i, c: (0, 0)),
            pl.BlockSpec((1, HID), lambda i, c: (0, 0)),
            pl.BlockSpec((1, HID), lambda i, c: (0, 0)),
            pl.BlockSpec((HID, 128), lambda i, c: (0, c)),
            pl.BlockSpec((2, NP), lambda i, c: (0, 0)),
        ],
        out_specs=[
            pl.BlockSpec((NBLK, HID), lambda i, c: (i, 0)),
            pl.BlockSpec((NBLK, 128), lambda i, c: (c * NG + i, 0)),
        ],
        out_shape=[
            jax.ShapeDtypeStruct((NP, HID), _f32),
            jax.ShapeDtypeStruct((2 * NP, 128), _f32),
        ],
    )(h, stats, g, be, W2, deg2)


def tc_final(h2, stats2, g2, be2, r1, x_p, Wi1, Wg1, Wo1, bi1, bg1, bo1,
             Wi2, Wg2, Wo2, bi2, bg2, bo2, W_lin, b_lin):
    def body(href, statref, gref, beref, rref, xref,
             wi1, wg1, wo1, b_i1, b_g1, b_o1,
             wi2, wg2, wo2, b_i2, b_g2, b_o2, wl, bl, oref):
        r2 = _bn_apply(href[...], statref[...], gref, beref)
        X = jnp.concatenate([rref[...], r2], axis=1)
        ii = jax.nn.sigmoid(jnp.dot(X, wi1[...], preferred_element_type=_f32) + b_i1[...])
        gg = jnp.tanh(jnp.dot(X, wg1[...], preferred_element_type=_f32) + b_g1[...])
        oo = jax.nn.sigmoid(jnp.dot(X, wo1[...], preferred_element_type=_f32) + b_o1[...])
        h1l = oo * jnp.tanh(ii * gg)
        ii2 = jax.nn.sigmoid(jnp.dot(h1l, wi2[...], preferred_element_type=_f32) + b_i2[...])
        gg2 = jnp.tanh(jnp.dot(h1l, wg2[...], preferred_element_type=_f32) + b_g2[...])
        oo2 = jax.nn.sigmoid(jnp.dot(h1l, wo2[...], preferred_element_type=_f32) + b_o2[...])
        h2l = oo2 * jnp.tanh(ii2 * gg2)
        Hc = jnp.concatenate([h1l, h2l, xref[...]], axis=1)
        He = jnp.where(Hc > 0, Hc, jnp.exp(jnp.minimum(Hc, 0.0)) - 1.0)
        oref[...] = jnp.dot(He, wl[...], preferred_element_type=_f32) + bl[...]

    whole = lambda shp: pl.BlockSpec(shp, lambda i: (0, 0))
    return pl.pallas_call(
        body,
        grid=(NG,),
        in_specs=[
            pl.BlockSpec((NBLK, HID), lambda i: (i, 0)),
            whole((2, HID)),
            whole((1, HID)),
            whole((1, HID)),
            pl.BlockSpec((NBLK, HID), lambda i: (i, 0)),
            pl.BlockSpec((NBLK, IN), lambda i: (i, 0)),
            whole((2 * HID, HID)), whole((2 * HID, HID)), whole((2 * HID, HID)),
            whole((1, HID)), whole((1, HID)), whole((1, HID)),
            whole((HID, HID)), whole((HID, HID)), whole((HID, HID)),
            whole((1, HID)), whole((1, HID)), whole((1, HID)),
            whole((2 * HID + IN, OUT)),
            whole((1, OUT)),
        ],
        out_specs=pl.BlockSpec((NBLK, OUT), lambda i: (i, 0)),
        out_shape=jax.ShapeDtypeStruct((NP, OUT), _f32),
    )(h2, stats2, g2, be2, r1, x_p, Wi1, Wg1, Wo1, bi1, bg1, bo1,
      Wi2, Wg2, Wo2, bi2, bg2, bo2, W_lin, b_lin)


def kernel(x, edge_index, edge_weight, W1, b1, g1, be1, W2, b2, g2, be2,
           w_ih1, w_hh1, b_ih1, b_hh1, w_ih2, w_hh2, b_ih2, b_hh2,
           W_lin, b_lin):
    pad_e = EPAD - E
    x_p = jnp.zeros((NP, IN), _f32).at[:N].set(x)
    src2d = jnp.concatenate([edge_index[0], jnp.zeros((pad_e,), _i32)]).reshape(EROWS, EW_)
    dst2d = jnp.concatenate([edge_index[1], jnp.zeros((pad_e,), _i32)]).reshape(EROWS, EW_)
    ew2d = jnp.concatenate([edge_weight, jnp.zeros((pad_e,), _f32)]).reshape(EROWS, EW_)

    b1r = b1.reshape(1, HID)
    g1r = g1.reshape(1, HID)
    be1r = be1.reshape(1, HID)
    b2r = b2.reshape(1, HID)
    g2r = g2.reshape(1, HID)
    be2r = be2.reshape(1, HID)
    wt1 = w_ih1.T
    Wi1, Wg1, Wo1 = wt1[:, 0:HID], wt1[:, 2 * HID:3 * HID], wt1[:, 3 * HID:]
    wt2 = w_ih2.T
    Wi2, Wg2, Wo2 = wt2[:, 0:HID], wt2[:, 2 * HID:3 * HID], wt2[:, 3 * HID:]
    bsum1 = (b_ih1 + b_hh1).reshape(1, 4 * HID)
    bi1, bg1, bo1 = bsum1[:, 0:HID], bsum1[:, 2 * HID:3 * HID], bsum1[:, 3 * HID:]
    bsum2 = (b_ih2 + b_hh2).reshape(1, 4 * HID)
    bi2, bg2, bo2 = bsum2[:, 0:HID], bsum2[:, 2 * HID:3 * HID], bsum2[:, 3 * HID:]
    b_linr = b_lin.reshape(1, OUT)

    deg2 = sc_degree(dst2d, ew2d)
    xs1 = tc_xw_scale(x_p, W1, deg2)
    s1 = sc_spmm(src2d, dst2d, ew2d, xs1)
    h1, st1 = tc_conv_post(s1, xs1, deg2, b1r)
    r1, xs2 = tc_bn_matmul(h1, st1, g1r, be1r, W2, deg2)
    s2 = sc_spmm(src2d, dst2d, ew2d, xs2)
    h2, st2 = tc_conv_post(s2, xs2, deg2, b2r)
    out = tc_final(h2, st2, g2r, be2r, r1, x_p, Wi1, Wg1, Wo1, bi1, bg1, bo1,
                   Wi2, Wg2, Wo2, bi2, bg2, bo2, W_lin, b_linr)
    return out[:N]


# speedup vs baseline: 4.1195x; 1.0956x over previous
import functools

import jax
import jax.numpy as jnp
from jax import lax
from jax.experimental import pallas as pl
from jax.experimental.pallas import tpu as pltpu
from jax.experimental.pallas import tpu_sc as plsc

N = 10062
E = 321984
IN = 128
HID = 256
OUT = 128
EPS = 1e-5

NP = 10240
NBLK = 128
NG = NP // NBLK
EW_ = 112
EROWS = 3072
EPAD = EROWS * EW_
TROWS = EROWS // 16
AROWS = EROWS // 32
NPT = NP // 16
CH = 16

_f32 = jnp.float32
_i32 = jnp.int32

_WCHUNKS = (112, 112, 112, 112, 112, 80)


def _zero16():
    return jnp.zeros((16,), _f32)


@functools.cache
def _sc_degree_call():
    mesh = plsc.VectorSubcoreMesh(core_axis_name="c", subcore_axis_name="s")
    return functools.partial(
        pl.kernel,
        mesh=mesh,
        out_type=jax.ShapeDtypeStruct((2, NP), _f32),
        scratch_types=[
            pltpu.VMEM((AROWS, EW_), _i32),
            pltpu.VMEM((AROWS, EW_), _f32),
            pltpu.VMEM((NPT,), _f32),
            pltpu.VMEM_SHARED((NP,), _f32),
        ],
    )(_sc_degree_body)


def _sc_degree_body(dst_hbm, ew_hbm, out_hbm, dst_v, ew_v, zv, acc_sh):
    c = lax.axis_index("c")
    s = lax.axis_index("s")

    def zb(i, carry):
        zv[pl.ds(i * 16, 16)] = _zero16()
        return carry

    lax.fori_loop(0, NPT // 16, zb, 0)
    pltpu.sync_copy(zv, acc_sh.at[pl.ds(s * NPT, NPT)])
    plsc.subcore_barrier()

    base = (c * 16 + s) * AROWS
    pltpu.sync_copy(dst_hbm.at[pl.ds(base, AROWS)], dst_v)
    pltpu.sync_copy(ew_hbm.at[pl.ds(base, AROWS)], ew_v)

    def body(g, carry):
        pltpu.sync_copy(ew_v.at[g], acc_sh.at[dst_v.at[g]], add=True)
        return carry

    lax.fori_loop(0, AROWS, body, 0)
    plsc.subcore_barrier()
    pltpu.sync_copy(acc_sh.at[pl.ds(s * NPT, NPT)], out_hbm.at[c, pl.ds(s * NPT, NPT)])


@functools.cache
def _sc_spmm_call():
    mesh = plsc.VectorSubcoreMesh(core_axis_name="c", subcore_axis_name="s")
    return functools.partial(
        pl.kernel,
        mesh=mesh,
        out_type=jax.ShapeDtypeStruct((2 * NP, 128), _f32),
        scratch_types=[
            pltpu.VMEM((CH, EW_), _i32),
            pltpu.VMEM((CH, EW_), _i32),
            pltpu.VMEM((CH, EW_), _f32),
            pltpu.VMEM((EW_, 128), _f32),
            pltpu.VMEM((EW_, 128), _f32),
            pltpu.VMEM_SHARED((NP, 128), _f32),
            pltpu.SemaphoreType.DMA,
            pltpu.SemaphoreType.DMA,
            pltpu.SemaphoreType.DMA,
            pltpu.SemaphoreType.DMA,
        ],
    )(_sc_spmm_body)


def _sc_spmm_body(src_hbm, dst_hbm, ew_hbm, xs_hbm, out_hbm,
                  src_v, dst_v, ew_v, rbuf0, rbuf1, acc_sh,
                  gs0, gs1, ss0, ss1):
    c = lax.axis_index("c")
    s = lax.axis_index("s")
    coff = c * NP
    rbufs = (rbuf0, rbuf1)
    gsems = (gs0, gs1)
    ssems = (ss0, ss1)

    def zb(i, carry):
        for f in range(8):
            rbuf0[i, pl.ds(f * 16, 16)] = _zero16()
        return carry

    lax.fori_loop(0, EW_, zb, 0)
    off = 0
    for sz in _WCHUNKS:
        pltpu.sync_copy(rbuf0.at[pl.ds(0, sz)],
                        acc_sh.at[pl.ds(s * NPT + off, sz)])
        off += sz
    plsc.subcore_barrier()

    base = s * TROWS

    def _mult(rb, g):
        for j in range(EW_ // 16):
            wrow = ew_v[g, pl.ds(j * 16, 16)]
            for l in range(16):
                i = j * 16 + l
                wv = wrow[l]
                for f in range(8):
                    rb[i, pl.ds(f * 16, 16)] = rb[i, pl.ds(f * 16, 16)] * wv

    def _issue_gather(b, g):
        pltpu.async_copy(xs_hbm.at[src_v.at[g]], rbufs[b], gsems[b])

    def _wait_gather(b):
        pltpu.make_async_copy(xs_hbm.at[pl.ds(0, EW_)], rbufs[b], gsems[b]).wait()

    def _issue_scatter(b, g):
        pltpu.async_copy(rbufs[b], acc_sh.at[dst_v.at[g]], ssems[b], add=True)

    def _wait_scatter(b):
        pltpu.make_async_copy(rbufs[b], acc_sh.at[pl.ds(0, EW_)], ssems[b]).wait()

    def outer(t, carry):
        row0 = base + t * CH
        pltpu.sync_copy(src_hbm.at[pl.ds(row0, CH)], src_v)
        pltpu.sync_copy(dst_hbm.at[pl.ds(row0, CH)], dst_v)
        pltpu.sync_copy(ew_hbm.at[pl.ds(row0, CH)], ew_v)

        def adj(g, carry2):
            for j in range(EW_ // 16):
                sl = src_v[g, pl.ds(j * 16, 16)]
                src_v[g, pl.ds(j * 16, 16)] = sl + coff
            return carry2

        lax.fori_loop(0, CH, adj, 0)

        _issue_gather(0, 0)
        _issue_gather(1, 1)

        def body(p, carry2):
            for b in range(2):
                g = p * 2 + b
                _wait_gather(b)
                _mult(rbufs[b], g)
                _issue_scatter(b, g)
            for b in range(2):
                _wait_scatter(b)

                @pl.when(p < CH // 2 - 1)
                def _():
                    _issue_gather(b, p * 2 + 2 + b)
            return carry2

        lax.fori_loop(0, CH // 2, body, 0)
        return carry

    lax.fori_loop(0, TROWS // CH, outer, 0)
    plsc.subcore_barrier()

    off = 0
    for sz in _WCHUNKS:
        pltpu.sync_copy(acc_sh.at[pl.ds(s * NPT + off, sz)],
                        out_hbm.at[pl.ds(coff + s * NPT + off, sz)])
        off += sz


def sc_degree(dst2d, ew2d):
    return _sc_degree_call()(dst2d, ew2d)


def sc_spmm(src2d, dst2d, ew2d, xs):
    return _sc_spmm_call()(src2d, dst2d, ew2d, xs)


def _dis_block(degref, i):
    d = degref[0, pl.ds(i * NBLK, NBLK)] + degref[1, pl.ds(i * NBLK, NBLK)] + 1.0
    return lax.rsqrt(d)


def tc_xw_scale(x_p, W, deg2):
    kdim = x_p.shape[1]

    def body(xref, wref, degref, oref):
        i = pl.program_id(0)
        dis = _dis_block(degref, i)
        xw = jnp.dot(xref[...], wref[...], preferred_element_type=_f32)
        oref[...] = xw * dis[:, None]

    return pl.pallas_call(
        body,
        grid=(NG, 2),
        in_specs=[
            pl.BlockSpec((NBLK, kdim), lambda i, c: (i, 0)),
            pl.BlockSpec((kdim, 128), lambda i, c: (0, c)),
            pl.BlockSpec((2, NP), lambda i, c: (0, 0)),
        ],
        out_specs=pl.BlockSpec((NBLK, 128), lambda i, c: (c * NG + i, 0)),
        out_shape=jax.ShapeDtypeStruct((2 * NP, 128), _f32),
    )(x_p, W, deg2)


def tc_conv_post(s_parts, xs_parts, deg2, b):
    def body(saref, sbref, xaref, xbref, degref, bref, href, statref, acc):
        i = pl.program_id(0)
        dis = _dis_block(degref, i)[:, None]
        ha = dis * (saref[...] + xaref[...]) + bref[:, :128]
        hb = dis * (sbref[...] + xbref[...]) + bref[:, 128:]
        h = jax.nn.relu(jnp.concatenate([ha, hb], axis=1))
        href[...] = h
        rid = i * NBLK + lax.broadcasted_iota(_i32, (NBLK, 1), 0)
        hm = jnp.where(rid < N, h, 0.0)
        blk = jnp.concatenate([jnp.sum(hm, 0, keepdims=True),
                               jnp.sum(hm * hm, 0, keepdims=True)], axis=0)

        @pl.when(i == 0)
        def _():
            acc[...] = jnp.zeros((2, HID), _f32)

        acc[...] += blk

        @pl.when(i == NG - 1)
        def _():
            statref[...] = acc[...]

    return pl.pallas_call(
        body,
        grid=(NG,),
        in_specs=[
            pl.BlockSpec((NBLK, 128), lambda i: (i, 0)),
            pl.BlockSpec((NBLK, 128), lambda i: (NG + i, 0)),
            pl.BlockSpec((NBLK, 128), lambda i: (i, 0)),
            pl.BlockSpec((NBLK, 128), lambda i: (NG + i, 0)),
            pl.BlockSpec((2, NP), lambda i: (0, 0)),
            pl.BlockSpec((1, HID), lambda i: (0, 0)),
        ],
        out_specs=[
            pl.BlockSpec((NBLK, HID), lambda i: (i, 0)),
            pl.BlockSpec((2, HID), lambda i: (0, 0)),
        ],
        out_shape=[
            jax.ShapeDtypeStruct((NP, HID), _f32),
            jax.ShapeDtypeStruct((2, HID), _f32),
        ],
        scratch_shapes=[pltpu.VMEM((2, HID), _f32)],
    )(s_parts, s_parts, xs_parts, xs_parts, deg2, b)


def _bn_apply(h, stats, gref, beref):
    mean = stats[0:1, :] * (1.0 / N)
    var = stats[1:2, :] * (1.0 / N) - mean * mean
    scale = gref[...] * lax.rsqrt(var + EPS)
    shift = beref[...] - mean * scale
    return h * scale + shift


def tc_bn_matmul(h, stats, g, be, W2, deg2):
    def body(href, statref, gref, beref, wref, degref, rref, oref):
        i = pl.program_id(0)
        r = _bn_apply(href[...], statref[...], gref, beref)
        rref[...] = r
        dis = _dis_block(degref, i)
        oref[...] = jnp.dot(r, wref[...], preferred_element_type=_f32) * dis[:, None]

    return pl.pallas_call(
        body,
        grid=(NG, 2),
        in_specs=[
            pl.BlockSpec((NBLK, HID), lambda i, c: (i, 0)),
            pl.BlockSpec((2, HID), lambda i, c: (0, 0)),
            pl.BlockSpec((1, HID), lambda i, c: (0, 0)),
            pl.BlockSpec((1, HID), lambda i, c: (0, 0)),
            pl.BlockSpec((HID, 128), lambda i, c: (0, c)),
            pl.BlockSpec((2, NP), lambda i, c: (0, 0)),
        ],
        out_specs=[
            pl.BlockSpec((NBLK, HID), lambda i, c: (i, 0)),
            pl.BlockSpec((NBLK, 128), lambda i, c: (c * NG + i, 0)),
        ],
        out_shape=[
            jax.ShapeDtypeStruct((NP, HID), _f32),
            jax.ShapeDtypeStruct((2 * NP, 128), _f32),
        ],
    )(h, stats, g, be, W2, deg2)


def tc_final(h2, stats2, g2, be2, r1, x_p, Wi1, Wg1, Wo1, bi1, bg1, bo1,
             Wi2, Wg2, Wo2, bi2, bg2, bo2, W_lin, b_lin):
    def body(href, statref, gref, beref, rref, xref,
             wi1, wg1, wo1, b_i1, b_g1, b_o1,
             wi2, wg2, wo2, b_i2, b_g2, b_o2, wl, bl, oref):
        r2 = _bn_apply(href[...], statref[...], gref, beref)
        X = jnp.concatenate([rref[...], r2], axis=1)
        ii = jax.nn.sigmoid(jnp.dot(X, wi1[...], preferred_element_type=_f32) + b_i1[...])
        gg = jnp.tanh(jnp.dot(X, wg1[...], preferred_element_type=_f32) + b_g1[...])
        oo = jax.nn.sigmoid(jnp.dot(X, wo1[...], preferred_element_type=_f32) + b_o1[...])
        h1l = oo * jnp.tanh(ii * gg)
        ii2 = jax.nn.sigmoid(jnp.dot(h1l, wi2[...], preferred_element_type=_f32) + b_i2[...])
        gg2 = jnp.tanh(jnp.dot(h1l, wg2[...], preferred_element_type=_f32) + b_g2[...])
        oo2 = jax.nn.sigmoid(jnp.dot(h1l, wo2[...], preferred_element_type=_f32) + b_o2[...])
        h2l = oo2 * jnp.tanh(ii2 * gg2)
        Hc = jnp.concatenate([h1l, h2l, xref[...]], axis=1)
        He = jnp.where(Hc > 0, Hc, jnp.exp(jnp.minimum(Hc, 0.0)) - 1.0)
        oref[...] = jnp.dot(He, wl[...], preferred_element_type=_f32) + bl[...]

    whole = lambda shp: pl.BlockSpec(shp, lambda i: (0, 0))
    return pl.pallas_call(
        body,
        grid=(NG,),
        in_specs=[
            pl.BlockSpec((NBLK, HID), lambda i: (i, 0)),
            whole((2, HID)),
            whole((1, HID)),
            whole((1, HID)),
            pl.BlockSpec((NBLK, HID), lambda i: (i, 0)),
            pl.BlockSpec((NBLK, IN), lambda i: (i, 0)),
            whole((2 * HID, HID)), whole((2 * HID, HID)), whole((2 * HID, HID)),
            whole((1, HID)), whole((1, HID)), whole((1, HID)),
            whole((HID, HID)), whole((HID, HID)), whole((HID, HID)),
            whole((1, HID)), whole((1, HID)), whole((1, HID)),
            whole((2 * HID + IN, OUT)),
            whole((1, OUT)),
        ],
        out_specs=pl.BlockSpec((NBLK, OUT), lambda i: (i, 0)),
        out_shape=jax.ShapeDtypeStruct((NP, OUT), _f32),
    )(h2, stats2, g2, be2, r1, x_p, Wi1, Wg1, Wo1, bi1, bg1, bo1,
      Wi2, Wg2, Wo2, bi2, bg2, bo2, W_lin, b_lin)


def kernel(x, edge_index, edge_weight, W1, b1, g1, be1, W2, b2, g2, be2,
           w_ih1, w_hh1, b_ih1, b_hh1, w_ih2, w_hh2, b_ih2, b_hh2,
           W_lin, b_lin):
    pad_e = EPAD - E
    x_p = jnp.zeros((NP, IN), _f32).at[:N].set(x)
    src2d = jnp.concatenate([edge_index[0], jnp.zeros((pad_e,), _i32)]).reshape(EROWS, EW_)
    dst2d = jnp.concatenate([edge_index[1], jnp.zeros((pad_e,), _i32)]).reshape(EROWS, EW_)
    ew2d = jnp.concatenate([edge_weight, jnp.zeros((pad_e,), _f32)]).reshape(EROWS, EW_)

    b1r = b1.reshape(1, HID)
    g1r = g1.reshape(1, HID)
    be1r = be1.reshape(1, HID)
    b2r = b2.reshape(1, HID)
    g2r = g2.reshape(1, HID)
    be2r = be2.reshape(1, HID)
    wt1 = w_ih1.T
    Wi1, Wg1, Wo1 = wt1[:, 0:HID], wt1[:, 2 * HID:3 * HID], wt1[:, 3 * HID:]
    wt2 = w_ih2.T
    Wi2, Wg2, Wo2 = wt2[:, 0:HID], wt2[:, 2 * HID:3 * HID], wt2[:, 3 * HID:]
    bsum1 = (b_ih1 + b_hh1).reshape(1, 4 * HID)
    bi1, bg1, bo1 = bsum1[:, 0:HID], bsum1[:, 2 * HID:3 * HID], bsum1[:, 3 * HID:]
    bsum2 = (b_ih2 + b_hh2).reshape(1, 4 * HID)
    bi2, bg2, bo2 = bsum2[:, 0:HID], bsum2[:, 2 * HID:3 * HID], bsum2[:, 3 * HID:]
    b_linr = b_lin.reshape(1, OUT)

    deg2 = sc_degree(dst2d, ew2d)
    xs1 = tc_xw_scale(x_p, W1, deg2)
    s1 = sc_spmm(src2d, dst2d, ew2d, xs1)
    h1, st1 = tc_conv_post(s1, xs1, deg2, b1r)
    r1, xs2 = tc_bn_matmul(h1, st1, g1r, be1r, W2, deg2)
    s2 = sc_spmm(src2d, dst2d, ew2d, xs2)
    h2, st2 = tc_conv_post(s2, xs2, deg2, b2r)
    out = tc_final(h2, st2, g2r, be2r, r1, x_p, Wi1, Wg1, Wo1, bi1, bg1, bo1,
                   Wi2, Wg2, Wo2, bi2, bg2, bo2, W_lin, b_linr)
    return out[:N]


# speedup vs baseline: 8.3922x; 2.0372x over previous
import functools

import jax
import jax.numpy as jnp
from jax import lax
from jax.experimental import pallas as pl
from jax.experimental.pallas import tpu as pltpu
from jax.experimental.pallas import tpu_sc as plsc

N = 10062
E = 321984
IN = 128
HID = 256
OUT = 128
EPS = 1e-5

NP = 10240
NBLK = 128
NG = NP // NBLK
EW_ = 112
EROWS = 3072
EPAD = EROWS * EW_
TROWS = EROWS // 16
AROWS = EROWS // 32
NPT = NP // 16
CH = 16
RING = 2
QW = 64

_f32 = jnp.float32
_i32 = jnp.int32

_WCHUNKS = (112, 112, 112, 112, 112, 80)


def _zero16():
    return jnp.zeros((16,), _f32)


@functools.cache
def _sc_degree_call():
    mesh = plsc.VectorSubcoreMesh(core_axis_name="c", subcore_axis_name="s")
    return functools.partial(
        pl.kernel,
        mesh=mesh,
        out_type=jax.ShapeDtypeStruct((2, NP), _f32),
        scratch_types=[
            pltpu.VMEM((AROWS, EW_), _i32),
            pltpu.VMEM((AROWS, EW_), _f32),
            pltpu.VMEM((NPT,), _f32),
            pltpu.VMEM_SHARED((NP,), _f32),
        ],
    )(_sc_degree_body)


def _sc_degree_body(dst_hbm, ew_hbm, out_hbm, dst_v, ew_v, zv, acc_sh):
    c = lax.axis_index("c")
    s = lax.axis_index("s")

    def zb(i, carry):
        zv[pl.ds(i * 16, 16)] = _zero16()
        return carry

    lax.fori_loop(0, NPT // 16, zb, 0)
    pltpu.sync_copy(zv, acc_sh.at[pl.ds(s * NPT, NPT)])
    plsc.subcore_barrier()

    base = (c * 16 + s) * AROWS
    pltpu.sync_copy(dst_hbm.at[pl.ds(base, AROWS)], dst_v)
    pltpu.sync_copy(ew_hbm.at[pl.ds(base, AROWS)], ew_v)

    def body(g, carry):
        pltpu.sync_copy(ew_v.at[g], acc_sh.at[dst_v.at[g]], add=True)
        return carry

    lax.fori_loop(0, AROWS, body, 0)
    plsc.subcore_barrier()
    pltpu.sync_copy(acc_sh.at[pl.ds(s * NPT, NPT)], out_hbm.at[c, pl.ds(s * NPT, NPT)])


@functools.cache
def _sc_spmm_call():
    mesh = plsc.VectorSubcoreMesh(core_axis_name="c", subcore_axis_name="s")
    return functools.partial(
        pl.kernel,
        mesh=mesh,
        compiler_params=pltpu.CompilerParams(use_tc_tiling_on_sc=False),
        out_type=jax.ShapeDtypeStruct((4 * NP, QW), _f32),
        scratch_types=[
            pltpu.VMEM((CH, EW_), _i32),
            pltpu.VMEM((CH, EW_), _i32),
            pltpu.VMEM((CH, EW_), _f32),
            pltpu.VMEM((EW_, QW), _f32),
            pltpu.VMEM((EW_, QW), _f32),
            pltpu.VMEM_SHARED((NP, QW), _f32),
            pltpu.VMEM_SHARED((NP, QW), _f32),
            pltpu.SemaphoreType.DMA,
            pltpu.SemaphoreType.DMA,
            pltpu.SemaphoreType.DMA,
            pltpu.SemaphoreType.DMA,
        ],
    )(_sc_spmm_body)


def _sc_spmm_body(src_hbm, dst_hbm, ew_hbm, xs_hbm, out_hbm,
                  src_v, dst_v, ew_v, rbuf0, rbuf1,
                  table_sh, acc_sh,
                  gs0, gs1, ss0, ss1):
    c = lax.axis_index("c")
    s = lax.axis_index("s")
    coff = c * NP
    rbufs = (rbuf0, rbuf1)
    gsems = (gs0, gs1)
    ssems = (ss0, ss1)
    base = s * TROWS
    nrow = s * NPT
    nv = QW // 16

    def _mult(rb, g):
        for j in range(EW_ // 16):
            wrow = ew_v[g, pl.ds(j * 16, 16)]
            for l in range(16):
                i = j * 16 + l
                wv = wrow[l]
                for f in range(nv):
                    rb[i, pl.ds(f * 16, 16)] = rb[i, pl.ds(f * 16, 16)] * wv

    def _issue_gather(b, g):
        pltpu.async_copy(table_sh.at[src_v.at[g]], rbufs[b], gsems[b])

    def _wait_gather(b):
        pltpu.make_async_copy(xs_hbm.at[pl.ds(0, EW_)], rbufs[b], gsems[b]).wait()

    def _issue_scatter(b, g):
        pltpu.async_copy(rbufs[b], acc_sh.at[dst_v.at[g]], ssems[b], add=True)

    def _wait_scatter(b):
        pltpu.make_async_copy(rbufs[b], acc_sh.at[pl.ds(0, EW_)], ssems[b]).wait()

    def one_pass(q, carry):
        qrow = (2 * c + q) * NP

        def zb(i, carry2):
            for f in range(nv):
                rbuf0[i, pl.ds(f * 16, 16)] = _zero16()
            return carry2

        lax.fori_loop(0, EW_, zb, 0)
        off = 0
        for sz in _WCHUNKS:
            pltpu.sync_copy(rbuf0.at[pl.ds(0, sz)],
                            acc_sh.at[pl.ds(nrow + off, sz)])
            off += sz
        off = 0
        for sz in _WCHUNKS:
            pltpu.sync_copy(xs_hbm.at[pl.ds(qrow + nrow + off, sz)],
                            rbuf1.at[pl.ds(0, sz)])
            pltpu.sync_copy(rbuf1.at[pl.ds(0, sz)],
                            table_sh.at[pl.ds(nrow + off, sz)])
            off += sz
        plsc.subcore_barrier()

        def outer(t, carry2):
            row0 = base + t * CH
            pltpu.sync_copy(src_hbm.at[pl.ds(row0, CH)], src_v)
            pltpu.sync_copy(dst_hbm.at[pl.ds(row0, CH)], dst_v)
            pltpu.sync_copy(ew_hbm.at[pl.ds(row0, CH)], ew_v)

            for b in range(RING):
                _issue_gather(b, b)

            def body(p, carry3):
                for b in range(RING):
                    g = p * RING + b
                    _wait_gather(b)
                    _mult(rbufs[b], g)
                    _issue_scatter(b, g)
                for b in range(RING):
                    _wait_scatter(b)

                    @pl.when(p < CH // RING - 1)
                    def _():
                        _issue_gather(b, p * RING + RING + b)
                return carry3

            lax.fori_loop(0, CH // RING, body, 0)
            return carry2

        lax.fori_loop(0, TROWS // CH, outer, 0)
        plsc.subcore_barrier()

        off = 0
        for sz in _WCHUNKS:
            pltpu.sync_copy(acc_sh.at[pl.ds(nrow + off, sz)],
                            out_hbm.at[pl.ds(qrow + nrow + off, sz)])
            off += sz
        return carry

    lax.fori_loop(0, 128 // QW, one_pass, 0)


def sc_degree(dst2d, ew2d):
    return _sc_degree_call()(dst2d, ew2d)


def sc_spmm(src2d, dst2d, ew2d, xs):
    return _sc_spmm_call()(src2d, dst2d, ew2d, xs)


def _dis_block(degref, i):
    d = degref[0, pl.ds(i * NBLK, NBLK)] + degref[1, pl.ds(i * NBLK, NBLK)] + 1.0
    return lax.rsqrt(d)


def tc_xw_scale(x_p, W, deg2):
    kdim = x_p.shape[1]

    Wq = jnp.stack([W[:, q * QW:(q + 1) * QW] for q in range(4)])

    def body(xref, wref, degref, oref):
        i = pl.program_id(0)
        dis = _dis_block(degref, i)
        xw = jnp.dot(xref[...], wref[0], preferred_element_type=_f32)
        oref[...] = xw * dis[:, None]

    return pl.pallas_call(
        body,
        grid=(NG, 4),
        in_specs=[
            pl.BlockSpec((NBLK, kdim), lambda i, q: (i, 0)),
            pl.BlockSpec((1, kdim, QW), lambda i, q: (q, 0, 0)),
            pl.BlockSpec((2, NP), lambda i, q: (0, 0)),
        ],
        out_specs=pl.BlockSpec((NBLK, QW), lambda i, q: (q * NG + i, 0)),
        out_shape=jax.ShapeDtypeStruct((4 * NP, QW), _f32),
    )(x_p, Wq, deg2)


def tc_conv_post(s_parts, xs_parts, deg2, b):
    def body(s0, s1, s2, s3, x0, x1, x2, x3, degref, bref, href, statref, acc):
        i = pl.program_id(0)
        dis = _dis_block(degref, i)[:, None]
        srefs = (s0, s1, s2, s3)
        xrefs = (x0, x1, x2, x3)
        parts = [dis * (srefs[q][...] + xrefs[q][...]) +
                 bref[:, q * QW:(q + 1) * QW] for q in range(4)]
        h = jax.nn.relu(jnp.concatenate(parts, axis=1))
        href[...] = h
        rid = i * NBLK + lax.broadcasted_iota(_i32, (NBLK, 1), 0)
        hm = jnp.where(rid < N, h, 0.0)
        blk = jnp.concatenate([jnp.sum(hm, 0, keepdims=True),
                               jnp.sum(hm * hm, 0, keepdims=True)], axis=0)

        @pl.when(i == 0)
        def _():
            acc[...] = jnp.zeros((2, HID), _f32)

        acc[...] += blk

        @pl.when(i == NG - 1)
        def _():
            statref[...] = acc[...]

    qspecs = [pl.BlockSpec((NBLK, QW), (lambda i, q=q: (q * NG + i, 0)))
              for q in range(4)]
    return pl.pallas_call(
        body,
        grid=(NG,),
        in_specs=qspecs + qspecs + [
            pl.BlockSpec((2, NP), lambda i: (0, 0)),
            pl.BlockSpec((1, HID), lambda i: (0, 0)),
        ],
        out_specs=[
            pl.BlockSpec((NBLK, HID), lambda i: (i, 0)),
            pl.BlockSpec((2, HID), lambda i: (0, 0)),
        ],
        out_shape=[
            jax.ShapeDtypeStruct((NP, HID), _f32),
            jax.ShapeDtypeStruct((2, HID), _f32),
        ],
        scratch_shapes=[pltpu.VMEM((2, HID), _f32)],
    )(*([s_parts] * 4), *([xs_parts] * 4), deg2, b)


def _bn_apply(h, stats, gref, beref):
    mean = stats[0:1, :] * (1.0 / N)
    var = stats[1:2, :] * (1.0 / N) - mean * mean
    scale = gref[...] * lax.rsqrt(var + EPS)
    shift = beref[...] - mean * scale
    return h * scale + shift


def tc_bn_matmul(h, stats, g, be, W2, deg2):
    Wq = jnp.stack([W2[:, q * QW:(q + 1) * QW] for q in range(4)])

    def body(href, statref, gref, beref, wref, degref, rref, oref):
        i = pl.program_id(0)
        r = _bn_apply(href[...], statref[...], gref, beref)
        rref[...] = r
        dis = _dis_block(degref, i)
        oref[...] = jnp.dot(r, wref[0], preferred_element_type=_f32) * dis[:, None]

    return pl.pallas_call(
        body,
        grid=(NG, 4),
        in_specs=[
            pl.BlockSpec((NBLK, HID), lambda i, q: (i, 0)),
            pl.BlockSpec((2, HID), lambda i, q: (0, 0)),
            pl.BlockSpec((1, HID), lambda i, q: (0, 0)),
            pl.BlockSpec((1, HID), lambda i, q: (0, 0)),
            pl.BlockSpec((1, HID, QW), lambda i, q: (q, 0, 0)),
            pl.BlockSpec((2, NP), lambda i, q: (0, 0)),
        ],
        out_specs=[
            pl.BlockSpec((NBLK, HID), lambda i, q: (i, 0)),
            pl.BlockSpec((NBLK, QW), lambda i, q: (q * NG + i, 0)),
        ],
        out_shape=[
            jax.ShapeDtypeStruct((NP, HID), _f32),
            jax.ShapeDtypeStruct((4 * NP, QW), _f32),
        ],
    )(h, stats, g, be, Wq, deg2)


def tc_final(h2, stats2, g2, be2, r1, x_p, Wi1, Wg1, Wo1, bi1, bg1, bo1,
             Wi2, Wg2, Wo2, bi2, bg2, bo2, W_lin, b_lin):
    def body(href, statref, gref, beref, rref, xref,
             wi1, wg1, wo1, b_i1, b_g1, b_o1,
             wi2, wg2, wo2, b_i2, b_g2, b_o2, wl, bl, oref):
        r2 = _bn_apply(href[...], statref[...], gref, beref)
        X = jnp.concatenate([rref[...], r2], axis=1)
        ii = jax.nn.sigmoid(jnp.dot(X, wi1[...], preferred_element_type=_f32) + b_i1[...])
        gg = jnp.tanh(jnp.dot(X, wg1[...], preferred_element_type=_f32) + b_g1[...])
        oo = jax.nn.sigmoid(jnp.dot(X, wo1[...], preferred_element_type=_f32) + b_o1[...])
        h1l = oo * jnp.tanh(ii * gg)
        ii2 = jax.nn.sigmoid(jnp.dot(h1l, wi2[...], preferred_element_type=_f32) + b_i2[...])
        gg2 = jnp.tanh(jnp.dot(h1l, wg2[...], preferred_element_type=_f32) + b_g2[...])
        oo2 = jax.nn.sigmoid(jnp.dot(h1l, wo2[...], preferred_element_type=_f32) + b_o2[...])
        h2l = oo2 * jnp.tanh(ii2 * gg2)
        Hc = jnp.concatenate([h1l, h2l, xref[...]], axis=1)
        He = jnp.where(Hc > 0, Hc, jnp.exp(jnp.minimum(Hc, 0.0)) - 1.0)
        oref[...] = jnp.dot(He, wl[...], preferred_element_type=_f32) + bl[...]

    whole = lambda shp: pl.BlockSpec(shp, lambda i: (0, 0))
    return pl.pallas_call(
        body,
        grid=(NG,),
        in_specs=[
            pl.BlockSpec((NBLK, HID), lambda i: (i, 0)),
            whole((2, HID)),
            whole((1, HID)),
            whole((1, HID)),
            pl.BlockSpec((NBLK, HID), lambda i: (i, 0)),
            pl.BlockSpec((NBLK, IN), lambda i: (i, 0)),
            whole((2 * HID, HID)), whole((2 * HID, HID)), whole((2 * HID, HID)),
            whole((1, HID)), whole((1, HID)), whole((1, HID)),
            whole((HID, HID)), whole((HID, HID)), whole((HID, HID)),
            whole((1, HID)), whole((1, HID)), whole((1, HID)),
            whole((2 * HID + IN, OUT)),
            whole((1, OUT)),
        ],
        out_specs=pl.BlockSpec((NBLK, OUT), lambda i: (i, 0)),
        out_shape=jax.ShapeDtypeStruct((NP, OUT), _f32),
    )(h2, stats2, g2, be2, r1, x_p, Wi1, Wg1, Wo1, bi1, bg1, bo1,
      Wi2, Wg2, Wo2, bi2, bg2, bo2, W_lin, b_lin)


def kernel(x, edge_index, edge_weight, W1, b1, g1, be1, W2, b2, g2, be2,
           w_ih1, w_hh1, b_ih1, b_hh1, w_ih2, w_hh2, b_ih2, b_hh2,
           W_lin, b_lin):
    pad_e = EPAD - E
    x_p = jnp.zeros((NP, IN), _f32).at[:N].set(x)
    src2d = jnp.concatenate([edge_index[0], jnp.zeros((pad_e,), _i32)]).reshape(EROWS, EW_)
    dst2d = jnp.concatenate([edge_index[1], jnp.zeros((pad_e,), _i32)]).reshape(EROWS, EW_)
    ew2d = jnp.concatenate([edge_weight, jnp.zeros((pad_e,), _f32)]).reshape(EROWS, EW_)

    b1r = b1.reshape(1, HID)
    g1r = g1.reshape(1, HID)
    be1r = be1.reshape(1, HID)
    b2r = b2.reshape(1, HID)
    g2r = g2.reshape(1, HID)
    be2r = be2.reshape(1, HID)
    wt1 = w_ih1.T
    Wi1, Wg1, Wo1 = wt1[:, 0:HID], wt1[:, 2 * HID:3 * HID], wt1[:, 3 * HID:]
    wt2 = w_ih2.T
    Wi2, Wg2, Wo2 = wt2[:, 0:HID], wt2[:, 2 * HID:3 * HID], wt2[:, 3 * HID:]
    bsum1 = (b_ih1 + b_hh1).reshape(1, 4 * HID)
    bi1, bg1, bo1 = bsum1[:, 0:HID], bsum1[:, 2 * HID:3 * HID], bsum1[:, 3 * HID:]
    bsum2 = (b_ih2 + b_hh2).reshape(1, 4 * HID)
    bi2, bg2, bo2 = bsum2[:, 0:HID], bsum2[:, 2 * HID:3 * HID], bsum2[:, 3 * HID:]
    b_linr = b_lin.reshape(1, OUT)

    deg2 = sc_degree(dst2d, ew2d)
    xs1 = tc_xw_scale(x_p, W1, deg2)
    s1 = sc_spmm(src2d, dst2d, ew2d, xs1)
    h1, st1 = tc_conv_post(s1, xs1, deg2, b1r)
    r1, xs2 = tc_bn_matmul(h1, st1, g1r, be1r, W2, deg2)
    s2 = sc_spmm(src2d, dst2d, ew2d, xs2)
    h2, st2 = tc_conv_post(s2, xs2, deg2, b2r)
    out = tc_final(h2, st2, g2r, be2r, r1, x_p, Wi1, Wg1, Wo1, bi1, bg1, bo1,
                   Wi2, Wg2, Wo2, bi2, bg2, bo2, W_lin, b_linr)
    return out[:N]


# speedup vs baseline: 8.4480x; 1.0066x over previous
import functools

import jax
import jax.numpy as jnp
from jax import lax
from jax.experimental import pallas as pl
from jax.experimental.pallas import tpu as pltpu
from jax.experimental.pallas import tpu_sc as plsc

N = 10062
E = 321984
IN = 128
HID = 256
OUT = 128
EPS = 1e-5

NP = 10240
NBLK = 128
NG = NP // NBLK
EW_ = 112
EROWS = 3072
EPAD = EROWS * EW_
TROWS = EROWS // 16
AROWS = EROWS // 32
NPT = NP // 16
CH = 16
RING = 2
QW = 64

_f32 = jnp.float32
_i32 = jnp.int32

_WCHUNKS = (112, 112, 112, 112, 112, 80)


def _zero16():
    return jnp.zeros((16,), _f32)


@functools.cache
def _sc_degree_call():
    mesh = plsc.VectorSubcoreMesh(core_axis_name="c", subcore_axis_name="s")
    return functools.partial(
        pl.kernel,
        mesh=mesh,
        out_type=jax.ShapeDtypeStruct((2, NP), _f32),
        scratch_types=[
            pltpu.VMEM((AROWS, EW_), _i32),
            pltpu.VMEM((AROWS, EW_), _f32),
            pltpu.VMEM((NPT,), _f32),
            pltpu.VMEM_SHARED((NP,), _f32),
        ],
    )(_sc_degree_body)


def _sc_degree_body(dst_hbm, ew_hbm, out_hbm, dst_v, ew_v, zv, acc_sh):
    c = lax.axis_index("c")
    s = lax.axis_index("s")

    def zb(i, carry):
        zv[pl.ds(i * 16, 16)] = _zero16()
        return carry

    lax.fori_loop(0, NPT // 16, zb, 0)
    pltpu.sync_copy(zv, acc_sh.at[pl.ds(s * NPT, NPT)])
    plsc.subcore_barrier()

    base = (c * 16 + s) * AROWS
    pltpu.sync_copy(dst_hbm.at[pl.ds(base, AROWS)], dst_v)
    pltpu.sync_copy(ew_hbm.at[pl.ds(base, AROWS)], ew_v)

    def body(g, carry):
        pltpu.sync_copy(ew_v.at[g], acc_sh.at[dst_v.at[g]], add=True)
        return carry

    lax.fori_loop(0, AROWS, body, 0)
    plsc.subcore_barrier()
    pltpu.sync_copy(acc_sh.at[pl.ds(s * NPT, NPT)], out_hbm.at[c, pl.ds(s * NPT, NPT)])


@functools.cache
def _sc_spmm_call():
    mesh = plsc.VectorSubcoreMesh(core_axis_name="c", subcore_axis_name="s")
    return functools.partial(
        pl.kernel,
        mesh=mesh,
        compiler_params=pltpu.CompilerParams(use_tc_tiling_on_sc=False),
        out_type=jax.ShapeDtypeStruct((4 * NP, QW), _f32),
        scratch_types=[
            pltpu.VMEM((CH, EW_), _i32),
            pltpu.VMEM((CH, EW_), _i32),
            pltpu.VMEM((CH, EW_), _f32),
            pltpu.VMEM((EW_, QW), _f32),
            pltpu.VMEM((EW_, QW), _f32),
            pltpu.VMEM_SHARED((NP, QW), _f32),
            pltpu.VMEM_SHARED((NP, QW), _f32),
            pltpu.SemaphoreType.DMA,
            pltpu.SemaphoreType.DMA,
            pltpu.SemaphoreType.DMA,
            pltpu.SemaphoreType.DMA,
        ],
    )(_sc_spmm_body)


def _sc_spmm_body(src_hbm, dst_hbm, ew_hbm, xs_hbm, out_hbm,
                  src_v, dst_v, ew_v, rbuf0, rbuf1,
                  table_sh, acc_sh,
                  gs0, gs1, ss0, ss1):
    c = lax.axis_index("c")
    s = lax.axis_index("s")
    coff = c * NP
    rbufs = (rbuf0, rbuf1)
    gsems = (gs0, gs1)
    ssems = (ss0, ss1)
    base = s * TROWS
    nrow = s * NPT
    nv = QW // 16

    def _mult(rb, g):
        for j in range(EW_ // 16):
            wrow = ew_v[g, pl.ds(j * 16, 16)]
            for l in range(16):
                i = j * 16 + l
                wv = wrow[l]
                for f in range(nv):
                    rb[i, pl.ds(f * 16, 16)] = rb[i, pl.ds(f * 16, 16)] * wv

    def _issue_gather(b, g):
        pltpu.async_copy(table_sh.at[src_v.at[g]], rbufs[b], gsems[b])

    def _wait_gather(b):
        pltpu.make_async_copy(xs_hbm.at[pl.ds(0, EW_)], rbufs[b], gsems[b]).wait()

    def _issue_scatter(b, g):
        pltpu.async_copy(rbufs[b], acc_sh.at[dst_v.at[g]], ssems[b], add=True)

    def _wait_scatter(b):
        pltpu.make_async_copy(rbufs[b], acc_sh.at[pl.ds(0, EW_)], ssems[b]).wait()

    def one_pass(q, carry):
        qrow = (2 * c + q) * NP

        def zb(i, carry2):
            for f in range(nv):
                rbuf0[i, pl.ds(f * 16, 16)] = _zero16()
            return carry2

        lax.fori_loop(0, EW_, zb, 0)
        off = 0
        for sz in _WCHUNKS:
            pltpu.sync_copy(rbuf0.at[pl.ds(0, sz)],
                            acc_sh.at[pl.ds(nrow + off, sz)])
            off += sz
        off = 0
        for sz in _WCHUNKS:
            pltpu.sync_copy(xs_hbm.at[pl.ds(qrow + nrow + off, sz)],
                            rbuf1.at[pl.ds(0, sz)])
            pltpu.sync_copy(rbuf1.at[pl.ds(0, sz)],
                            table_sh.at[pl.ds(nrow + off, sz)])
            off += sz
        plsc.subcore_barrier()

        def outer(t, carry2):
            row0 = base + t * CH
            pltpu.sync_copy(src_hbm.at[pl.ds(row0, CH)], src_v)
            pltpu.sync_copy(dst_hbm.at[pl.ds(row0, CH)], dst_v)
            pltpu.sync_copy(ew_hbm.at[pl.ds(row0, CH)], ew_v)

            for b in range(RING):
                _issue_gather(b, b)

            def body(p, carry3):
                for b in range(RING):
                    g = p * RING + b
                    _wait_gather(b)
                    _mult(rbufs[b], g)
                    _issue_scatter(b, g)
                for b in range(RING):
                    _wait_scatter(b)

                    @pl.when(p < CH // RING - 1)
                    def _():
                        _issue_gather(b, p * RING + RING + b)
                return carry3

            lax.fori_loop(0, CH // RING, body, 0)
            return carry2

        lax.fori_loop(0, TROWS // CH, outer, 0)
        plsc.subcore_barrier()

        off = 0
        for sz in _WCHUNKS:
            pltpu.sync_copy(acc_sh.at[pl.ds(nrow + off, sz)],
                            out_hbm.at[pl.ds(qrow + nrow + off, sz)])
            off += sz
        return carry

    lax.fori_loop(0, 128 // QW, one_pass, 0)


def sc_degree(dst2d, ew2d):
    return _sc_degree_call()(dst2d, ew2d)


def sc_spmm(src2d, dst2d, ew2d, xs):
    return _sc_spmm_call()(src2d, dst2d, ew2d, xs)


def _dis_block(degref, i):
    d = degref[0, pl.ds(i * NBLK, NBLK)] + degref[1, pl.ds(i * NBLK, NBLK)] + 1.0
    return lax.rsqrt(d)


def tc_xw_scale(x_p, W, deg2):
    kdim = x_p.shape[1]

    Wq = jnp.stack([W[:, q * QW:(q + 1) * QW] for q in range(4)]).astype(jnp.bfloat16)

    def body(xref, wref, degref, oref):
        i = pl.program_id(0)
        dis = _dis_block(degref, i)
        xw = jnp.dot(xref[...].astype(jnp.bfloat16), wref[0],
                     preferred_element_type=_f32)
        oref[...] = xw * dis[:, None]

    return pl.pallas_call(
        body,
        grid=(NG, 4),
        in_specs=[
            pl.BlockSpec((NBLK, kdim), lambda i, q: (i, 0)),
            pl.BlockSpec((1, kdim, QW), lambda i, q: (q, 0, 0)),
            pl.BlockSpec((2, NP), lambda i, q: (0, 0)),
        ],
        out_specs=pl.BlockSpec((NBLK, QW), lambda i, q: (q * NG + i, 0)),
        out_shape=jax.ShapeDtypeStruct((4 * NP, QW), _f32),
    )(x_p, Wq, deg2)


def tc_conv_post(s_parts, xs_parts, deg2, b):
    def body(s0, s1, s2, s3, x0, x1, x2, x3, degref, bref, href, statref, acc):
        i = pl.program_id(0)
        dis = _dis_block(degref, i)[:, None]
        srefs = (s0, s1, s2, s3)
        xrefs = (x0, x1, x2, x3)
        parts = [dis * (srefs[q][...] + xrefs[q][...]) +
                 bref[:, q * QW:(q + 1) * QW] for q in range(4)]
        h = jax.nn.relu(jnp.concatenate(parts, axis=1))
        href[...] = h
        rid = i * NBLK + lax.broadcasted_iota(_i32, (NBLK, 1), 0)
        hm = jnp.where(rid < N, h, 0.0)
        blk = jnp.concatenate([jnp.sum(hm, 0, keepdims=True),
                               jnp.sum(hm * hm, 0, keepdims=True)], axis=0)

        @pl.when(i == 0)
        def _():
            acc[...] = jnp.zeros((2, HID), _f32)

        acc[...] += blk

        @pl.when(i == NG - 1)
        def _():
            statref[...] = acc[...]

    qspecs = [pl.BlockSpec((NBLK, QW), (lambda i, q=q: (q * NG + i, 0)))
              for q in range(4)]
    return pl.pallas_call(
        body,
        grid=(NG,),
        in_specs=qspecs + qspecs + [
            pl.BlockSpec((2, NP), lambda i: (0, 0)),
            pl.BlockSpec((1, HID), lambda i: (0, 0)),
        ],
        out_specs=[
            pl.BlockSpec((NBLK, HID), lambda i: (i, 0)),
            pl.BlockSpec((2, HID), lambda i: (0, 0)),
        ],
        out_shape=[
            jax.ShapeDtypeStruct((NP, HID), _f32),
            jax.ShapeDtypeStruct((2, HID), _f32),
        ],
        scratch_shapes=[pltpu.VMEM((2, HID), _f32)],
    )(*([s_parts] * 4), *([xs_parts] * 4), deg2, b)


def _bn_apply(h, stats, gref, beref):
    mean = stats[0:1, :] * (1.0 / N)
    var = stats[1:2, :] * (1.0 / N) - mean * mean
    scale = gref[...] * lax.rsqrt(var + EPS)
    shift = beref[...] - mean * scale
    return h * scale + shift


def tc_bn_matmul(h, stats, g, be, W2, deg2):
    Wq = jnp.stack([W2[:, q * QW:(q + 1) * QW] for q in range(4)]).astype(jnp.bfloat16)

    def body(href, statref, gref, beref, wref, degref, rref, oref):
        i = pl.program_id(0)
        r = _bn_apply(href[...], statref[...], gref, beref)
        rref[...] = r
        dis = _dis_block(degref, i)
        oref[...] = jnp.dot(r.astype(jnp.bfloat16), wref[0],
                            preferred_element_type=_f32) * dis[:, None]

    return pl.pallas_call(
        body,
        grid=(NG, 4),
        in_specs=[
            pl.BlockSpec((NBLK, HID), lambda i, q: (i, 0)),
            pl.BlockSpec((2, HID), lambda i, q: (0, 0)),
            pl.BlockSpec((1, HID), lambda i, q: (0, 0)),
            pl.BlockSpec((1, HID), lambda i, q: (0, 0)),
            pl.BlockSpec((1, HID, QW), lambda i, q: (q, 0, 0)),
            pl.BlockSpec((2, NP), lambda i, q: (0, 0)),
        ],
        out_specs=[
            pl.BlockSpec((NBLK, HID), lambda i, q: (i, 0)),
            pl.BlockSpec((NBLK, QW), lambda i, q: (q * NG + i, 0)),
        ],
        out_shape=[
            jax.ShapeDtypeStruct((NP, HID), _f32),
            jax.ShapeDtypeStruct((4 * NP, QW), _f32),
        ],
    )(h, stats, g, be, Wq, deg2)


def tc_final(h2, stats2, g2, be2, r1, x_p, Wi1, Wg1, Wo1, bi1, bg1, bo1,
             Wi2, Wg2, Wo2, bi2, bg2, bo2, W_lin, b_lin):
    def body(href, statref, gref, beref, rref, xref,
             wi1, wg1, wo1, b_i1, b_g1, b_o1,
             wi2, wg2, wo2, b_i2, b_g2, b_o2, wl, bl, oref):
        r2 = _bn_apply(href[...], statref[...], gref, beref)
        X = jnp.concatenate([rref[...], r2], axis=1).astype(jnp.bfloat16)
        ii = jax.nn.sigmoid(jnp.dot(X, wi1[...], preferred_element_type=_f32) + b_i1[...])
        gg = jnp.tanh(jnp.dot(X, wg1[...], preferred_element_type=_f32) + b_g1[...])
        oo = jax.nn.sigmoid(jnp.dot(X, wo1[...], preferred_element_type=_f32) + b_o1[...])
        h1l = oo * jnp.tanh(ii * gg)
        h1b = h1l.astype(jnp.bfloat16)
        ii2 = jax.nn.sigmoid(jnp.dot(h1b, wi2[...], preferred_element_type=_f32) + b_i2[...])
        gg2 = jnp.tanh(jnp.dot(h1b, wg2[...], preferred_element_type=_f32) + b_g2[...])
        oo2 = jax.nn.sigmoid(jnp.dot(h1b, wo2[...], preferred_element_type=_f32) + b_o2[...])
        h2l = oo2 * jnp.tanh(ii2 * gg2)
        Hc = jnp.concatenate([h1l, h2l, xref[...]], axis=1)
        He = jnp.where(Hc > 0, Hc, jnp.exp(jnp.minimum(Hc, 0.0)) - 1.0)
        oref[...] = jnp.dot(He.astype(jnp.bfloat16), wl[...],
                            preferred_element_type=_f32) + bl[...]

    whole = lambda shp: pl.BlockSpec(shp, lambda i: (0, 0))
    return pl.pallas_call(
        body,
        grid=(NG,),
        in_specs=[
            pl.BlockSpec((NBLK, HID), lambda i: (i, 0)),
            whole((2, HID)),
            whole((1, HID)),
            whole((1, HID)),
            pl.BlockSpec((NBLK, HID), lambda i: (i, 0)),
            pl.BlockSpec((NBLK, IN), lambda i: (i, 0)),
            whole((2 * HID, HID)), whole((2 * HID, HID)), whole((2 * HID, HID)),
            whole((1, HID)), whole((1, HID)), whole((1, HID)),
            whole((HID, HID)), whole((HID, HID)), whole((HID, HID)),
            whole((1, HID)), whole((1, HID)), whole((1, HID)),
            whole((2 * HID + IN, OUT)),
            whole((1, OUT)),
        ],
        out_specs=pl.BlockSpec((NBLK, OUT), lambda i: (i, 0)),
        out_shape=jax.ShapeDtypeStruct((NP, OUT), _f32),
    )(h2, stats2, g2, be2, r1, x_p, Wi1, Wg1, Wo1, bi1, bg1, bo1,
      Wi2, Wg2, Wo2, bi2, bg2, bo2, W_lin, b_lin)


def kernel(x, edge_index, edge_weight, W1, b1, g1, be1, W2, b2, g2, be2,
           w_ih1, w_hh1, b_ih1, b_hh1, w_ih2, w_hh2, b_ih2, b_hh2,
           W_lin, b_lin):
    pad_e = EPAD - E
    x_p = jnp.zeros((NP, IN), _f32).at[:N].set(x)
    src2d = jnp.concatenate([edge_index[0], jnp.zeros((pad_e,), _i32)]).reshape(EROWS, EW_)
    dst2d = jnp.concatenate([edge_index[1], jnp.zeros((pad_e,), _i32)]).reshape(EROWS, EW_)
    ew2d = jnp.concatenate([edge_weight, jnp.zeros((pad_e,), _f32)]).reshape(EROWS, EW_)

    b1r = b1.reshape(1, HID)
    g1r = g1.reshape(1, HID)
    be1r = be1.reshape(1, HID)
    b2r = b2.reshape(1, HID)
    g2r = g2.reshape(1, HID)
    be2r = be2.reshape(1, HID)
    bf16 = jnp.bfloat16
    wt1 = w_ih1.T.astype(bf16)
    Wi1, Wg1, Wo1 = wt1[:, 0:HID], wt1[:, 2 * HID:3 * HID], wt1[:, 3 * HID:]
    wt2 = w_ih2.T.astype(bf16)
    Wi2, Wg2, Wo2 = wt2[:, 0:HID], wt2[:, 2 * HID:3 * HID], wt2[:, 3 * HID:]
    bsum1 = (b_ih1 + b_hh1).reshape(1, 4 * HID)
    bi1, bg1, bo1 = bsum1[:, 0:HID], bsum1[:, 2 * HID:3 * HID], bsum1[:, 3 * HID:]
    bsum2 = (b_ih2 + b_hh2).reshape(1, 4 * HID)
    bi2, bg2, bo2 = bsum2[:, 0:HID], bsum2[:, 2 * HID:3 * HID], bsum2[:, 3 * HID:]
    b_linr = b_lin.reshape(1, OUT)
    W_linb = W_lin.astype(bf16)

    deg2 = sc_degree(dst2d, ew2d)
    xs1 = tc_xw_scale(x_p, W1, deg2)
    s1 = sc_spmm(src2d, dst2d, ew2d, xs1)
    h1, st1 = tc_conv_post(s1, xs1, deg2, b1r)
    r1, xs2 = tc_bn_matmul(h1, st1, g1r, be1r, W2, deg2)
    s2 = sc_spmm(src2d, dst2d, ew2d, xs2)
    h2, st2 = tc_conv_post(s2, xs2, deg2, b2r)
    out = tc_final(h2, st2, g2r, be2r, r1, x_p, Wi1, Wg1, Wo1, bi1, bg1, bo1,
                   Wi2, Wg2, Wo2, bi2, bg2, bo2, W_linb, b_linr)
    return out[:N]
